# Initial kernel scaffold; baseline (speedup 1.0000x reference)
import jax
import jax.numpy as jnp
from jax.experimental import pallas as pl


def kernel(mem, idx, val, W1, W2, b2, W3, b3):
    raise NotImplementedError("write your pallas kernel here")


# speedup vs baseline: 1.6625x; 1.6625x over previous
import functools

import jax
import jax.numpy as jnp
from jax import lax
from jax.experimental import pallas as pl
from jax.experimental.pallas import tpu as pltpu
from jax.experimental.pallas import tpu_sc as plsc

_M, _B, _D = 524288, 131072, 64
_H1, _H2, _NCLS = 128, 64, 3


def _head_t_body(x_ref, w1_ref, w2_ref, b2_ref, w3_ref, b3t_ref, o0_ref, o1_ref, o2_ref):
    x = x_ref[...]
    h = jnp.maximum(jnp.dot(x, w1_ref[...], preferred_element_type=jnp.float32), 0.0)
    h = jnp.maximum(
        jnp.dot(h, w2_ref[...], preferred_element_type=jnp.float32) + b2_ref[...], 0.0
    )
    ot = lax.dot_general(
        w3_ref[...], h, (((0,), (1,)), ((), ())), preferred_element_type=jnp.float32
    )
    ot = jnp.tanh(ot + b3t_ref[...])
    o0_ref[...] = ot[0]
    o1_ref[...] = ot[1]
    o2_ref[...] = ot[2]


def _head_t(x, W1, W2, b2, W3, b3, bm):
    n = x.shape[0]
    return pl.pallas_call(
        _head_t_body,
        grid=(n // bm,),
        in_specs=[
            pl.BlockSpec((bm, _D), lambda i: (i, 0)),
            pl.BlockSpec((_D, _H1), lambda i: (0, 0)),
            pl.BlockSpec((_H1, _H2), lambda i: (0, 0)),
            pl.BlockSpec((1, _H2), lambda i: (0, 0)),
            pl.BlockSpec((_H2, _NCLS), lambda i: (0, 0)),
            pl.BlockSpec((_NCLS, 1), lambda i: (0, 0)),
        ],
        out_specs=[pl.BlockSpec((bm,), lambda i: (i,)) for _ in range(_NCLS)],
        out_shape=[jax.ShapeDtypeStruct((n,), jnp.float32) for _ in range(_NCLS)],
    )(x, W1, W2, b2.reshape(1, _H2), W3, b3.reshape(_NCLS, 1))


_PLANE_PART = _M // 16
_SC_CHUNK = 4096
_N_SC_CHUNKS = _B // _SC_CHUNK


def _scatter_sc(mh, vh, idx):
    mesh = plsc.VectorSubcoreMesh(core_axis_name="c", subcore_axis_name="s")

    @functools.partial(
        pl.kernel,
        out_type=[jax.ShapeDtypeStruct((_M,), jnp.float32) for _ in range(_NCLS)],
        mesh=mesh,
        scratch_types=[
            pltpu.VMEM((_SC_CHUNK,), jnp.int32),
            pltpu.VMEM((_SC_CHUNK,), jnp.float32),
        ]
        + [pltpu.VMEM_SHARED((_M,), jnp.float32) for _ in range(_NCLS)],
    )
    def k(mh0, mh1, mh2, vh0, vh1, vh2, idx_hbm, o0, o1, o2, idx_v, data_v, p0, p1, p2):
        cid = lax.axis_index("c")
        sid = lax.axis_index("s")
        mhs, vhs, outs, planes = (mh0, mh1, mh2), (vh0, vh1, vh2), (o0, o1, o2), (p0, p1, p2)

        @pl.when(cid == 0)
        def _stage_in():
            off = sid * _PLANE_PART
            for k3 in range(_NCLS):
                pltpu.sync_copy(
                    mhs[k3].at[pl.ds(off, _PLANE_PART)],
                    planes[k3].at[pl.ds(off, _PLANE_PART)],
                )

        plsc.subcore_barrier()

        for k3 in range(_NCLS):

            @pl.when(jnp.logical_and(cid == 0, sid == k3))
            def _scatter(k3=k3):
                def chunk(c, carry):
                    off = c * _SC_CHUNK
                    pltpu.sync_copy(idx_hbm.at[pl.ds(off, _SC_CHUNK)], idx_v)
                    pltpu.sync_copy(vhs[k3].at[pl.ds(off, _SC_CHUNK)], data_v)
                    pltpu.sync_copy(data_v, planes[k3].at[idx_v])
                    return carry

                lax.fori_loop(0, _N_SC_CHUNKS, chunk, 0)

        plsc.subcore_barrier()

        @pl.when(cid == 0)
        def _stage_out():
            off = sid * _PLANE_PART
            for k3 in range(_NCLS):
                pltpu.sync_copy(
                    planes[k3].at[pl.ds(off, _PLANE_PART)],
                    outs[k3].at[pl.ds(off, _PLANE_PART)],
                )

    return k(mh[0], mh[1], mh[2], vh[0], vh[1], vh[2], idx)


def _pack_body(p0_ref, p1_ref, p2_ref, o_ref):
    o_ref[...] = jnp.stack([p0_ref[...], p1_ref[...], p2_ref[...]], axis=1)


def _pack(planes, bm):
    return pl.pallas_call(
        _pack_body,
        grid=(_M // bm,),
        in_specs=[pl.BlockSpec((bm,), lambda i: (i,)) for _ in range(_NCLS)],
        out_specs=pl.BlockSpec((bm, _NCLS), lambda i: (i, 0)),
        out_shape=jax.ShapeDtypeStruct((_M, _NCLS), jnp.float32),
    )(*planes)


def kernel(mem, idx, val, W1, W2, b2, W3, b3):
    mh = _head_t(mem, W1, W2, b2, W3, b3, bm=4096)
    vh = _head_t(val, W1, W2, b2, W3, b3, bm=4096)
    out_planes = _scatter_sc(mh, vh, idx)
    return _pack(out_planes, bm=4096)


# speedup vs baseline: 1.8844x; 1.1335x over previous
import functools

import jax
import jax.numpy as jnp
from jax import lax
from jax.experimental import pallas as pl
from jax.experimental.pallas import tpu as pltpu
from jax.experimental.pallas import tpu_sc as plsc

_M, _B, _D = 524288, 131072, 64
_H1, _H2, _NCLS = 128, 64, 3


def _head_t_body(x_ref, w1_ref, w2_ref, b2_ref, w3_ref, b3t_ref, o0_ref, o1_ref, o2_ref):
    x = x_ref[...]
    h = jnp.maximum(jnp.dot(x, w1_ref[...], preferred_element_type=jnp.float32), 0.0)
    h = jnp.maximum(
        jnp.dot(h, w2_ref[...], preferred_element_type=jnp.float32) + b2_ref[...], 0.0
    )
    ot = lax.dot_general(
        w3_ref[...], h, (((0,), (1,)), ((), ())), preferred_element_type=jnp.float32
    )
    ot = jnp.tanh(ot + b3t_ref[...])
    o0_ref[...] = ot[0]
    o1_ref[...] = ot[1]
    o2_ref[...] = ot[2]


def _head_t(x, W1, W2, b2, W3, b3, bm):
    n = x.shape[0]
    return pl.pallas_call(
        _head_t_body,
        grid=(n // bm,),
        in_specs=[
            pl.BlockSpec((bm, _D), lambda i: (i, 0)),
            pl.BlockSpec((_D, _H1), lambda i: (0, 0)),
            pl.BlockSpec((_H1, _H2), lambda i: (0, 0)),
            pl.BlockSpec((1, _H2), lambda i: (0, 0)),
            pl.BlockSpec((_H2, _NCLS), lambda i: (0, 0)),
            pl.BlockSpec((_NCLS, 1), lambda i: (0, 0)),
        ],
        out_specs=[pl.BlockSpec((bm,), lambda i: (i,)) for _ in range(_NCLS)],
        out_shape=[jax.ShapeDtypeStruct((n,), jnp.float32) for _ in range(_NCLS)],
    )(x, W1, W2, b2.reshape(1, _H2), W3, b3.reshape(_NCLS, 1))


_PLANE_PART = _M // 16
_SC_CHUNK = 16384
_N_SC_CHUNKS = _B // _SC_CHUNK


def _scatter_sc(mh, vh, idx):
    mesh = plsc.VectorSubcoreMesh(core_axis_name="c", subcore_axis_name="s")

    @functools.partial(
        pl.kernel,
        out_type=[jax.ShapeDtypeStruct((_M,), jnp.float32) for _ in range(_NCLS)],
        mesh=mesh,
        scratch_types=[
            pltpu.VMEM((_SC_CHUNK,), jnp.int32),
            pltpu.VMEM((_SC_CHUNK,), jnp.float32),
        ]
        + [pltpu.VMEM_SHARED((_M,), jnp.float32) for _ in range(_NCLS)],
    )
    def k(mh0, mh1, mh2, vh0, vh1, vh2, idx_hbm, o0, o1, o2, idx_v, data_v, p0, p1, p2):
        cid = lax.axis_index("c")
        sid = lax.axis_index("s")
        mhs, vhs, outs, planes = (mh0, mh1, mh2), (vh0, vh1, vh2), (o0, o1, o2), (p0, p1, p2)

        @pl.when(cid == 0)
        def _stage_in():
            off = sid * _PLANE_PART
            for k3 in range(_NCLS):
                pltpu.sync_copy(
                    mhs[k3].at[pl.ds(off, _PLANE_PART)],
                    planes[k3].at[pl.ds(off, _PLANE_PART)],
                )

        plsc.subcore_barrier()

        for k3 in range(_NCLS):

            @pl.when(jnp.logical_and(cid == 0, sid == k3))
            def _scatter(k3=k3):
                def chunk(c, carry):
                    off = c * _SC_CHUNK
                    pltpu.sync_copy(idx_hbm.at[pl.ds(off, _SC_CHUNK)], idx_v)
                    pltpu.sync_copy(vhs[k3].at[pl.ds(off, _SC_CHUNK)], data_v)
                    pltpu.sync_copy(data_v, planes[k3].at[idx_v])
                    return carry

                lax.fori_loop(0, _N_SC_CHUNKS, chunk, 0)

        plsc.subcore_barrier()

        @pl.when(cid == 0)
        def _stage_out():
            off = sid * _PLANE_PART
            for k3 in range(_NCLS):
                pltpu.sync_copy(
                    planes[k3].at[pl.ds(off, _PLANE_PART)],
                    outs[k3].at[pl.ds(off, _PLANE_PART)],
                )

    return k(mh[0], mh[1], mh[2], vh[0], vh[1], vh[2], idx)


def _pack_body(p0_ref, p1_ref, p2_ref, o_ref):
    xt = jnp.stack([p0_ref[...], p1_ref[...], p2_ref[...]], axis=0)
    r = lax.broadcasted_iota(jnp.int32, (_NCLS, _NCLS), 0)
    c = lax.broadcasted_iota(jnp.int32, (_NCLS, _NCLS), 1)
    eye = (r == c).astype(jnp.float32)
    o_ref[...] = lax.dot_general(
        xt, eye, (((0,), (0,)), ((), ())), preferred_element_type=jnp.float32
    )


def _pack(planes, bm):
    return pl.pallas_call(
        _pack_body,
        grid=(_M // bm,),
        in_specs=[pl.BlockSpec((bm,), lambda i: (i,)) for _ in range(_NCLS)],
        out_specs=pl.BlockSpec((bm, _NCLS), lambda i: (i, 0)),
        out_shape=jax.ShapeDtypeStruct((_M, _NCLS), jnp.float32),
    )(*planes)


def kernel(mem, idx, val, W1, W2, b2, W3, b3):
    mh = _head_t(mem, W1, W2, b2, W3, b3, bm=4096)
    vh = _head_t(val, W1, W2, b2, W3, b3, bm=4096)
    out_planes = _scatter_sc(mh, vh, idx)
    return _pack(out_planes, bm=4096)
